# Initial kernel scaffold; baseline (speedup 1.0000x reference)
import functools

import jax
import jax.numpy as jnp
from jax import lax
from jax.experimental import pallas as pl
from jax.experimental.pallas import tpu as pltpu
from jax.experimental.pallas import tpu_sc as plsc

N_NODES = 100000
N_EDGES = 6400000
N_GRAPHS = 64

NC = 2
NS = 16
NW = NC * NS
EPW = N_EDGES // NW
CHUNK = 800
IROW = 100
ROWS = CHUNK // IROW
NCHUNK = EPW // CHUNK
STEPS = CHUNK // 16
LAT = N_GRAPHS * 9


def _sc_body(tab_hbm, dst_hbm, src_hbm, shift_hbm, lat_hbm, out_hbm,
             dstv, srcv, rowd, rows_, shiftv, latv, outv, sem):
    wid = lax.axis_index("s") * NC + lax.axis_index("c")
    pltpu.sync_copy(lat_hbm, latv)

    iota = lax.iota(jnp.int32, 16)
    c0 = jnp.zeros((16,), jnp.int32)
    c1 = jnp.full((16,), 1, jnp.int32)
    c2 = jnp.full((16,), 2, jnp.int32)
    c3 = jnp.full((16,), 3, jnp.int32)

    def chunk_body(k, carry):
        base = wid * EPW + k * CHUNK
        r0 = base // IROW
        pltpu.sync_copy(dst_hbm.at[pl.ds(r0, ROWS)], dstv)
        pltpu.sync_copy(src_hbm.at[pl.ds(r0, ROWS)], srcv)
        pltpu.sync_copy(shift_hbm.at[pl.ds(base, CHUNK)], shiftv)
        copies = []
        for j in range(ROWS):
            cd = pltpu.make_async_copy(
                tab_hbm.at[dstv.at[j]], rowd.at[pl.ds(j * IROW, IROW)], sem)
            cs = pltpu.make_async_copy(
                tab_hbm.at[srcv.at[j]], rows_.at[pl.ds(j * IROW, IROW)], sem)
            cd.start()
            cs.start()
            copies.append(cd)
            copies.append(cs)
        for c in copies:
            c.wait()

        def step(i, carry2):
            ids = i * 16 + iota
            pdx = plsc.load_gather(rowd, [ids, c0])
            pdy = plsc.load_gather(rowd, [ids, c1])
            pdz = plsc.load_gather(rowd, [ids, c2])
            psx = plsc.load_gather(rows_, [ids, c0])
            psy = plsc.load_gather(rows_, [ids, c1])
            psz = plsc.load_gather(rows_, [ids, c2])
            bf = plsc.load_gather(rows_, [ids, c3])
            b9 = plsc.bitcast(bf, jnp.int32) * 9
            shx = plsc.load_gather(shiftv, [ids, c0])
            shy = plsc.load_gather(shiftv, [ids, c1])
            shz = plsc.load_gather(shiftv, [ids, c2])
            l00 = plsc.load_gather(latv, [b9])
            l01 = plsc.load_gather(latv, [b9 + 1])
            l02 = plsc.load_gather(latv, [b9 + 2])
            l10 = plsc.load_gather(latv, [b9 + 3])
            l11 = plsc.load_gather(latv, [b9 + 4])
            l12 = plsc.load_gather(latv, [b9 + 5])
            l20 = plsc.load_gather(latv, [b9 + 6])
            l21 = plsc.load_gather(latv, [b9 + 7])
            l22 = plsc.load_gather(latv, [b9 + 8])
            vx = pdx - psx + shx * l00 + shy * l10 + shz * l20
            vy = pdy - psy + shx * l01 + shy * l11 + shz * l21
            vz = pdz - psz + shx * l02 + shy * l12 + shz * l22
            x = jnp.maximum(vx * vx + vy * vy + vz * vz, 1e-30)
            ii = 0x5F3759DF - lax.shift_right_logical(
                plsc.bitcast(x, jnp.int32), 1)
            r = plsc.bitcast(ii, jnp.float32)
            r = r * (1.5 - 0.5 * x * r * r)
            r = r * (1.5 - 0.5 * x * r * r)
            r = r * (1.5 - 0.5 * x * r * r)
            plsc.store_scatter(outv, [ids], x * r)
            return carry2

        lax.fori_loop(0, STEPS, step, 0)
        pltpu.sync_copy(outv, out_hbm.at[pl.ds(base, CHUNK)])
        return carry

    lax.fori_loop(0, NCHUNK, chunk_body, 0)


_sc_call = pl.kernel(
    _sc_body,
    out_type=jax.ShapeDtypeStruct((N_EDGES,), jnp.float32),
    mesh=plsc.VectorSubcoreMesh(
        core_axis_name="c", subcore_axis_name="s",
        num_cores=NC, num_subcores=NS),
    scratch_types=[
        pltpu.VMEM((ROWS, IROW), jnp.int32),
        pltpu.VMEM((ROWS, IROW), jnp.int32),
        pltpu.VMEM((CHUNK, 4), jnp.float32),
        pltpu.VMEM((CHUNK, 4), jnp.float32),
        pltpu.VMEM((CHUNK, 3), jnp.float32),
        pltpu.VMEM((LAT,), jnp.float32),
        pltpu.VMEM((CHUNK,), jnp.float32),
        pltpu.SemaphoreType.DMA,
    ],
)


def kernel(pos, edge_index, edge_shift, lattice, batch_idx):
    tab = jnp.concatenate(
        [pos, lax.bitcast_convert_type(batch_idx, jnp.float32)[:, None]],
        axis=1)
    dst2d = edge_index[0].reshape(N_EDGES // IROW, IROW)
    src2d = edge_index[1].reshape(N_EDGES // IROW, IROW)
    lat_flat = lattice.reshape(LAT)
    return _sc_call(tab, dst2d, src2d, edge_shift, lat_flat)


# speedup vs baseline: 17.9570x; 17.9570x over previous
import jax
import jax.numpy as jnp
from jax import lax
from jax.experimental import pallas as pl
from jax.experimental.pallas import tpu as pltpu
from jax.experimental.pallas import tpu_sc as plsc

N_NODES = 100000
N_EDGES = 6400000
N_GRAPHS = 64

NC = 2
NS = 16
NW = NC * NS
EPW = N_EDGES // NW
CHUNK = 800
IROW = 80
ROWS = CHUNK // IROW
NCHUNK = EPW // CHUNK
STEPS = CHUNK // 16
LAT = N_GRAPHS * 9
NBUF = 7


def _sc_body(px_hbm, py_hbm, pz_hbm, bf_hbm, dst_hbm, src_hbm, shift_hbm,
             lat_hbm, out_hbm, *scr):
    sets = []
    per = 3 + NBUF + 1
    for s in range(2):
        blk = scr[s * per:(s + 1) * per]
        sets.append(dict(dstv=blk[0], srcv=blk[1], shiftv=blk[2],
                         bufs=blk[3:3 + NBUF], outv=blk[3 + NBUF]))
    latv = scr[2 * per]
    lsem = scr[2 * per + 1: 2 * per + 3]
    gsem = scr[2 * per + 3: 2 * per + 5]
    osem = scr[2 * per + 5: 2 * per + 7]

    wid = lax.axis_index("s") * NC + lax.axis_index("c")
    pltpu.sync_copy(lat_hbm, latv)
    iota = lax.iota(jnp.int32, 16)

    def chunk_base(k):
        return pl.multiple_of(wid * EPW + k * CHUNK, CHUNK)

    def lin_start(k, s):
        base = chunk_base(k)
        pltpu.make_async_copy(
            dst_hbm.at[pl.ds(base, CHUNK)], sets[s]["dstv"], lsem[s]).start()
        pltpu.make_async_copy(
            src_hbm.at[pl.ds(base, CHUNK)], sets[s]["srcv"], lsem[s]).start()
        pltpu.make_async_copy(
            shift_hbm.at[pl.ds(base * 3, CHUNK * 3)], sets[s]["shiftv"],
            lsem[s]).start()

    def lin_wait(s):
        pltpu.make_async_copy(
            dst_hbm.at[pl.ds(0, CHUNK)], sets[s]["dstv"], lsem[s]).wait()
        pltpu.make_async_copy(
            src_hbm.at[pl.ds(0, CHUNK)], sets[s]["srcv"], lsem[s]).wait()
        pltpu.make_async_copy(
            shift_hbm.at[pl.ds(0, CHUNK * 3)], sets[s]["shiftv"],
            lsem[s]).wait()

    def gather_start(s):
        st = sets[s]
        for j in range(ROWS):
            sl = pl.ds(j * IROW, IROW)
            dj = st["dstv"].at[sl]
            sj = st["srcv"].at[sl]
            for tab, buf in ((px_hbm, st["bufs"][0]),
                             (py_hbm, st["bufs"][1]),
                             (pz_hbm, st["bufs"][2])):
                pltpu.make_async_copy(tab.at[dj], buf.at[sl], gsem[s]).start()
            for tab, buf in ((px_hbm, st["bufs"][3]),
                             (py_hbm, st["bufs"][4]),
                             (pz_hbm, st["bufs"][5]),
                             (bf_hbm, st["bufs"][6])):
                pltpu.make_async_copy(tab.at[sj], buf.at[sl], gsem[s]).start()

    def gather_wait(s):
        st = sets[s]
        for j in range(ROWS):
            sl = pl.ds(j * IROW, IROW)
            dj = st["dstv"].at[sl]
            sj = st["srcv"].at[sl]
            for tab, buf in ((px_hbm, st["bufs"][0]),
                             (py_hbm, st["bufs"][1]),
                             (pz_hbm, st["bufs"][2])):
                pltpu.make_async_copy(tab.at[dj], buf.at[sl], gsem[s]).wait()
            for tab, buf in ((px_hbm, st["bufs"][3]),
                             (py_hbm, st["bufs"][4]),
                             (pz_hbm, st["bufs"][5]),
                             (bf_hbm, st["bufs"][6])):
                pltpu.make_async_copy(tab.at[sj], buf.at[sl], gsem[s]).wait()

    def out_start(k, s):
        base = chunk_base(k)
        pltpu.make_async_copy(
            sets[s]["outv"], out_hbm.at[pl.ds(base, CHUNK)], osem[s]).start()

    def out_wait(s):
        pltpu.make_async_copy(
            sets[s]["outv"], out_hbm.at[pl.ds(0, CHUNK)], osem[s]).wait()

    def compute(s):
        st = sets[s]
        pdx_v, pdy_v, pdz_v, psx_v, psy_v, psz_v, bfv = st["bufs"]
        shiftv = st["shiftv"]
        outv = st["outv"]

        def step(i, carry2):
            off = i * 16
            sl16 = pl.ds(off, 16)
            ids3 = (off + iota) * 3
            pdx = pdx_v[sl16]
            pdy = pdy_v[sl16]
            pdz = pdz_v[sl16]
            psx = psx_v[sl16]
            psy = psy_v[sl16]
            psz = psz_v[sl16]
            b9 = lax.bitcast_convert_type(bfv[sl16], jnp.int32) * 9
            shx = plsc.load_gather(shiftv, [ids3])
            shy = plsc.load_gather(shiftv, [ids3 + 1])
            shz = plsc.load_gather(shiftv, [ids3 + 2])
            l00 = plsc.load_gather(latv, [b9])
            l01 = plsc.load_gather(latv, [b9 + 1])
            l02 = plsc.load_gather(latv, [b9 + 2])
            l10 = plsc.load_gather(latv, [b9 + 3])
            l11 = plsc.load_gather(latv, [b9 + 4])
            l12 = plsc.load_gather(latv, [b9 + 5])
            l20 = plsc.load_gather(latv, [b9 + 6])
            l21 = plsc.load_gather(latv, [b9 + 7])
            l22 = plsc.load_gather(latv, [b9 + 8])
            vx = pdx - psx + shx * l00 + shy * l10 + shz * l20
            vy = pdy - psy + shx * l01 + shy * l11 + shz * l21
            vz = pdz - psz + shx * l02 + shy * l12 + shz * l22
            x = jnp.maximum(vx * vx + vy * vy + vz * vz, 1e-30)
            ii = 0x5F3759DF - lax.shift_right_logical(
                lax.bitcast_convert_type(x, jnp.int32), 1)
            r = lax.bitcast_convert_type(ii, jnp.float32)
            r = r * (1.5 - 0.5 * x * r * r)
            r = r * (1.5 - 0.5 * x * r * r)
            r = r * (1.5 - 0.5 * x * r * r)
            outv[sl16] = x * r
            return carry2

        lax.fori_loop(0, STEPS, step, 0)

    lin_start(0, 0)
    lin_start(1, 1)
    lin_wait(0)
    gather_start(0)

    def pair_body(kk, carry):
        k0 = kk * 2
        gather_wait(0)

        @pl.when(kk >= 1)
        def _():
            out_wait(0)

        lin_wait(1)
        gather_start(1)
        compute(0)
        out_start(k0, 0)

        @pl.when(kk < (NCHUNK // 2 - 1))
        def _():
            lin_start(k0 + 2, 0)

        k1 = k0 + 1
        gather_wait(1)

        @pl.when(kk >= 1)
        def _():
            out_wait(1)

        @pl.when(kk < (NCHUNK // 2 - 1))
        def _():
            lin_wait(0)
            gather_start(0)

        compute(1)
        out_start(k1, 1)

        @pl.when(kk < (NCHUNK // 2 - 1))
        def _():
            lin_start(k1 + 2, 1)

        return carry

    lax.fori_loop(0, NCHUNK // 2, pair_body, 0)
    out_wait(0)
    out_wait(1)


def _make_scratch():
    one_set = [
        pltpu.VMEM((CHUNK,), jnp.int32),
        pltpu.VMEM((CHUNK,), jnp.int32),
        pltpu.VMEM((CHUNK * 3,), jnp.float32),
    ] + [pltpu.VMEM((CHUNK,), jnp.float32) for _ in range(NBUF)] + [
        pltpu.VMEM((CHUNK,), jnp.float32),
    ]
    return (one_set * 2
            + [pltpu.VMEM((LAT,), jnp.float32)]
            + [pltpu.SemaphoreType.DMA] * 6)


_sc_call = pl.kernel(
    _sc_body,
    out_type=jax.ShapeDtypeStruct((N_EDGES,), jnp.float32),
    mesh=plsc.VectorSubcoreMesh(
        core_axis_name="c", subcore_axis_name="s",
        num_cores=NC, num_subcores=NS),
    compiler_params=pltpu.CompilerParams(needs_layout_passes=False),
    scratch_types=_make_scratch(),
)


def kernel(pos, edge_index, edge_shift, lattice, batch_idx):
    px = pos[:, 0]
    py = pos[:, 1]
    pz = pos[:, 2]
    bf = lax.bitcast_convert_type(batch_idx, jnp.float32)
    lat_flat = lattice.reshape(LAT)
    shift_flat = edge_shift.reshape(N_EDGES * 3)
    return _sc_call(px, py, pz, bf, edge_index[0], edge_index[1],
                    shift_flat, lat_flat)


# speedup vs baseline: 194.2648x; 10.8183x over previous
import jax
import jax.numpy as jnp
from jax import lax
from jax.experimental import pallas as pl
from jax.experimental.pallas import tpu as pltpu
from jax.experimental.pallas import tpu_sc as plsc

N_NODES = 100000
N_EDGES = 6400000
N_GRAPHS = 64

NC = 2
NS = 16
NW = NC * NS
EPW = N_EDGES // NW
CHUNK = 800
IROW = 80
ROWS = CHUNK // IROW
NCHUNK = EPW // CHUNK
STEPS = CHUNK // 16
LAT = N_GRAPHS * 9
NBUF = 7


def _sc_body(px_hbm, py_hbm, pz_hbm, bf_hbm, dst_hbm, src_hbm,
             shx_hbm, shy_hbm, shz_hbm, lat_hbm, out_hbm, *scr):
    sets = []
    per = 5 + NBUF + 1
    for s in range(2):
        blk = scr[s * per:(s + 1) * per]
        sets.append(dict(dstv=blk[0], srcv=blk[1], shxv=blk[2], shyv=blk[3],
                         shzv=blk[4], bufs=blk[5:5 + NBUF],
                         outv=blk[5 + NBUF]))
    latv = scr[2 * per]
    lsem = scr[2 * per + 1: 2 * per + 3]
    gsem = scr[2 * per + 3: 2 * per + 5]
    osem = scr[2 * per + 5: 2 * per + 7]
    px_s, py_s, pz_s, bf_s = scr[2 * per + 7: 2 * per + 11]

    sid = lax.axis_index("s")
    wid = sid * NC + lax.axis_index("c")
    pltpu.sync_copy(lat_hbm, latv)

    @pl.when(sid == 0)
    def _():
        pltpu.sync_copy(px_hbm, px_s)
        pltpu.sync_copy(py_hbm, py_s)
        pltpu.sync_copy(pz_hbm, pz_s)
        pltpu.sync_copy(bf_hbm, bf_s)

    plsc.subcore_barrier()
    iota = lax.iota(jnp.int32, 16)

    def chunk_base(k):
        return pl.multiple_of(wid * EPW + k * CHUNK, CHUNK)

    def lin_start(k, s):
        base = chunk_base(k)
        pltpu.make_async_copy(
            dst_hbm.at[pl.ds(base, CHUNK)], sets[s]["dstv"], lsem[s]).start()
        pltpu.make_async_copy(
            src_hbm.at[pl.ds(base, CHUNK)], sets[s]["srcv"], lsem[s]).start()
        pltpu.make_async_copy(
            shx_hbm.at[pl.ds(base, CHUNK)], sets[s]["shxv"], lsem[s]).start()
        pltpu.make_async_copy(
            shy_hbm.at[pl.ds(base, CHUNK)], sets[s]["shyv"], lsem[s]).start()
        pltpu.make_async_copy(
            shz_hbm.at[pl.ds(base, CHUNK)], sets[s]["shzv"], lsem[s]).start()

    def lin_wait(s):
        pltpu.make_async_copy(
            dst_hbm.at[pl.ds(0, CHUNK)], sets[s]["dstv"], lsem[s]).wait()
        pltpu.make_async_copy(
            src_hbm.at[pl.ds(0, CHUNK)], sets[s]["srcv"], lsem[s]).wait()
        pltpu.make_async_copy(
            shx_hbm.at[pl.ds(0, CHUNK)], sets[s]["shxv"], lsem[s]).wait()
        pltpu.make_async_copy(
            shy_hbm.at[pl.ds(0, CHUNK)], sets[s]["shyv"], lsem[s]).wait()
        pltpu.make_async_copy(
            shz_hbm.at[pl.ds(0, CHUNK)], sets[s]["shzv"], lsem[s]).wait()

    def gather_start(s):
        st = sets[s]
        for j in range(ROWS):
            sl = pl.ds(j * IROW, IROW)
            dj = st["dstv"].at[sl]
            sj = st["srcv"].at[sl]
            for tab, buf in ((px_s, st["bufs"][0]),
                             (py_s, st["bufs"][1]),
                             (pz_s, st["bufs"][2])):
                pltpu.make_async_copy(tab.at[dj], buf.at[sl], gsem[s]).start()
            for tab, buf in ((px_s, st["bufs"][3]),
                             (py_s, st["bufs"][4]),
                             (pz_s, st["bufs"][5]),
                             (bf_s, st["bufs"][6])):
                pltpu.make_async_copy(tab.at[sj], buf.at[sl], gsem[s]).start()

    def gather_wait(s):
        st = sets[s]
        for j in range(ROWS):
            sl = pl.ds(j * IROW, IROW)
            dj = st["dstv"].at[sl]
            sj = st["srcv"].at[sl]
            for tab, buf in ((px_s, st["bufs"][0]),
                             (py_s, st["bufs"][1]),
                             (pz_s, st["bufs"][2])):
                pltpu.make_async_copy(tab.at[dj], buf.at[sl], gsem[s]).wait()
            for tab, buf in ((px_s, st["bufs"][3]),
                             (py_s, st["bufs"][4]),
                             (pz_s, st["bufs"][5]),
                             (bf_s, st["bufs"][6])):
                pltpu.make_async_copy(tab.at[sj], buf.at[sl], gsem[s]).wait()

    def out_start(k, s):
        base = chunk_base(k)
        pltpu.make_async_copy(
            sets[s]["outv"], out_hbm.at[pl.ds(base, CHUNK)], osem[s]).start()

    def out_wait(s):
        pltpu.make_async_copy(
            sets[s]["outv"], out_hbm.at[pl.ds(0, CHUNK)], osem[s]).wait()

    def compute(s):
        st = sets[s]
        pdx_v, pdy_v, pdz_v, psx_v, psy_v, psz_v, bfv = st["bufs"]
        shxv = st["shxv"]
        shyv = st["shyv"]
        shzv = st["shzv"]
        outv = st["outv"]

        def sub(off):
            sl16 = pl.ds(off, 16)
            pdx = pdx_v[sl16]
            pdy = pdy_v[sl16]
            pdz = pdz_v[sl16]
            psx = psx_v[sl16]
            psy = psy_v[sl16]
            psz = psz_v[sl16]
            b9 = lax.bitcast_convert_type(bfv[sl16], jnp.int32) * 9
            shx = shxv[sl16]
            shy = shyv[sl16]
            shz = shzv[sl16]
            l00 = plsc.load_gather(latv, [b9])
            l01 = plsc.load_gather(latv, [b9 + 1])
            l02 = plsc.load_gather(latv, [b9 + 2])
            l10 = plsc.load_gather(latv, [b9 + 3])
            l11 = plsc.load_gather(latv, [b9 + 4])
            l12 = plsc.load_gather(latv, [b9 + 5])
            l20 = plsc.load_gather(latv, [b9 + 6])
            l21 = plsc.load_gather(latv, [b9 + 7])
            l22 = plsc.load_gather(latv, [b9 + 8])
            vx = pdx - psx + shx * l00 + shy * l10 + shz * l20
            vy = pdy - psy + shx * l01 + shy * l11 + shz * l21
            vz = pdz - psz + shx * l02 + shy * l12 + shz * l22
            x = jnp.maximum(vx * vx + vy * vy + vz * vz, 1e-30)
            ii = 0x5F3759DF - lax.shift_right_logical(
                lax.bitcast_convert_type(x, jnp.int32), 1)
            r = lax.bitcast_convert_type(ii, jnp.float32)
            r = r * (1.5 - 0.5 * x * r * r)
            r = r * (1.5 - 0.5 * x * r * r)
            outv[sl16] = x * r

        def step(i, carry2):
            off = i * 32
            sub(off)
            sub(off + 16)
            return carry2

        lax.fori_loop(0, STEPS // 2, step, 0)

    lin_start(0, 0)
    lin_start(1, 1)
    lin_wait(0)
    gather_start(0)

    def pair_body(kk, carry):
        k0 = kk * 2
        gather_wait(0)

        @pl.when(kk >= 1)
        def _():
            out_wait(0)

        lin_wait(1)
        gather_start(1)
        compute(0)
        out_start(k0, 0)

        @pl.when(kk < (NCHUNK // 2 - 1))
        def _():
            lin_start(k0 + 2, 0)

        k1 = k0 + 1
        gather_wait(1)

        @pl.when(kk >= 1)
        def _():
            out_wait(1)

        @pl.when(kk < (NCHUNK // 2 - 1))
        def _():
            lin_wait(0)
            gather_start(0)

        compute(1)
        out_start(k1, 1)

        @pl.when(kk < (NCHUNK // 2 - 1))
        def _():
            lin_start(k1 + 2, 1)

        return carry

    lax.fori_loop(0, NCHUNK // 2, pair_body, 0)
    out_wait(0)
    out_wait(1)


def _make_scratch():
    one_set = [
        pltpu.VMEM((CHUNK,), jnp.int32),
        pltpu.VMEM((CHUNK,), jnp.int32),
        pltpu.VMEM((CHUNK,), jnp.float32),
        pltpu.VMEM((CHUNK,), jnp.float32),
        pltpu.VMEM((CHUNK,), jnp.float32),
    ] + [pltpu.VMEM((CHUNK,), jnp.float32) for _ in range(NBUF)] + [
        pltpu.VMEM((CHUNK,), jnp.float32),
    ]
    return (one_set * 2
            + [pltpu.VMEM((LAT,), jnp.float32)]
            + [pltpu.SemaphoreType.DMA] * 6
            + [pltpu.VMEM_SHARED((N_NODES,), jnp.float32)
               for _ in range(4)])


_sc_call = pl.kernel(
    _sc_body,
    out_type=jax.ShapeDtypeStruct((N_EDGES,), jnp.float32),
    mesh=plsc.VectorSubcoreMesh(
        core_axis_name="c", subcore_axis_name="s",
        num_cores=NC, num_subcores=NS),
    compiler_params=pltpu.CompilerParams(needs_layout_passes=False),
    scratch_types=_make_scratch(),
)


def kernel(pos, edge_index, edge_shift, lattice, batch_idx):
    px = pos[:, 0]
    py = pos[:, 1]
    pz = pos[:, 2]
    bf = lax.bitcast_convert_type(batch_idx, jnp.float32)
    lat_flat = lattice.reshape(LAT)
    return _sc_call(px, py, pz, bf, edge_index[0], edge_index[1],
                    edge_shift[:, 0], edge_shift[:, 1], edge_shift[:, 2],
                    lat_flat)
